# baseline (speedup 1.0000x reference)
import jax
import jax.numpy as jnp
from jax.experimental import pallas as pl
from jax.experimental.pallas import tpu as pltpu

_IN = 4096
_OUT = 4096
_CB = 256

_BN_DQ = 256
_BM = 512
_BNO = 2048


def _dequant_body(cb_ref, idx_ref, x_ref, w_ref, xb_ref):
    cb = cb_ref[...]
    idx = idx_ref[...]
    lo = cb[:, :128]
    hi = cb[:, 128:]
    glo = jnp.take_along_axis(lo, jnp.bitwise_and(idx, 127), axis=1)
    ghi = jnp.take_along_axis(hi, jnp.bitwise_and(idx, 127), axis=1)
    w = jnp.where(idx < 128, glo, ghi)
    w_ref[...] = w.astype(jnp.bfloat16)
    xb_ref[...] = x_ref[...].astype(jnp.bfloat16)


def _matmul_body(x_ref, w_ref, b_ref, o_ref):
    acc = jax.lax.dot_general(
        x_ref[...], w_ref[...], (((1,), (1,)), ((), ())),
        preferred_element_type=jnp.float32)
    o_ref[...] = acc + b_ref[...]


def kernel(x, codebook, bias, indices):
    idx = indices.astype(jnp.int32)
    b, s, _ = x.shape
    m = b * s
    xm = x.reshape(m, _IN)

    n_dq = _OUT // _BN_DQ
    n_dq_half = n_dq // 2
    bx = m // n_dq
    w, xb = pl.pallas_call(
        _dequant_body,
        grid=(2, n_dq_half),
        in_specs=[
            pl.BlockSpec((_BN_DQ, _CB), lambda c, n: (c * n_dq_half + n, 0)),
            pl.BlockSpec((_BN_DQ, _IN), lambda c, n: (c * n_dq_half + n, 0)),
            pl.BlockSpec((bx, _IN), lambda c, n: (c * n_dq_half + n, 0)),
        ],
        out_specs=[
            pl.BlockSpec((_BN_DQ, _IN), lambda c, n: (c * n_dq_half + n, 0)),
            pl.BlockSpec((bx, _IN), lambda c, n: (c * n_dq_half + n, 0)),
        ],
        out_shape=[
            jax.ShapeDtypeStruct((_OUT, _IN), jnp.bfloat16),
            jax.ShapeDtypeStruct((m, _IN), jnp.bfloat16),
        ],
        compiler_params=pltpu.CompilerParams(
            dimension_semantics=("parallel", "arbitrary")),
        name="dequant_codebook",
    )(codebook, idx, xm)

    m_tiles = m // _BM // 2
    n_tiles = _OUT // _BNO

    out = pl.pallas_call(
        _matmul_body,
        grid=(2, n_tiles, m_tiles),
        in_specs=[
            pl.BlockSpec((_BM, _IN), lambda c, n, mm: (c * m_tiles + mm, 0)),
            pl.BlockSpec((_BNO, _IN), lambda c, n, mm: (n, 0)),
            pl.BlockSpec((1, _BNO), lambda c, n, mm: (0, n)),
        ],
        out_specs=pl.BlockSpec((_BM, _BNO), lambda c, n, mm: (c * m_tiles + mm, n)),
        out_shape=jax.ShapeDtypeStruct((m, _OUT), jnp.float32),
        compiler_params=pltpu.CompilerParams(
            dimension_semantics=("parallel", "arbitrary", "arbitrary")),
        name="dequant_linear_matmul",
    )(xb, w, bias.reshape(1, _OUT))
    return out.reshape(b, s, _OUT)


# speedup vs baseline: 1.0753x; 1.0753x over previous
import jax
import jax.numpy as jnp
from jax.experimental import pallas as pl
from jax.experimental.pallas import tpu as pltpu

_IN = 4096
_OUT = 4096
_CB = 256

_BN_DQ = 512
_BM = 256


def _dequant_body(cb_ref, idx_ref, w_ref):
    cb = cb_ref[...]
    idx = idx_ref[...]
    lo = cb[:, :128]
    hi = cb[:, 128:]
    idx7 = jnp.bitwise_and(idx, 127)
    glo = jnp.take_along_axis(lo, idx7, axis=1)
    ghi = jnp.take_along_axis(hi, idx7, axis=1)
    w = jnp.where(idx < 128, glo, ghi)
    w_ref[...] = w.astype(jnp.bfloat16)


def _matmul_body(x_ref, w_hbm, b_ref, o_ref, w_vmem, sem):
    @pl.when(pl.program_id(1) == 0)
    def _():
        pltpu.make_async_copy(w_hbm, w_vmem, sem).start()
        pltpu.make_async_copy(w_hbm, w_vmem, sem).wait()

    x = x_ref[...].astype(jnp.bfloat16)
    acc = jax.lax.dot_general(
        x, w_vmem[...], (((1,), (1,)), ((), ())),
        preferred_element_type=jnp.float32)
    o_ref[...] = acc + b_ref[...]


def kernel(x, codebook, bias, indices):
    idx = indices.astype(jnp.int32)

    n_dq = _OUT // _BN_DQ
    n_dq_half = n_dq // 2
    w = pl.pallas_call(
        _dequant_body,
        grid=(2, n_dq_half),
        in_specs=[
            pl.BlockSpec((_BN_DQ, _CB), lambda c, n: (c * n_dq_half + n, 0)),
            pl.BlockSpec((_BN_DQ, _IN), lambda c, n: (c * n_dq_half + n, 0)),
        ],
        out_specs=pl.BlockSpec((_BN_DQ, _IN), lambda c, n: (c * n_dq_half + n, 0)),
        out_shape=jax.ShapeDtypeStruct((_OUT, _IN), jnp.bfloat16),
        compiler_params=pltpu.CompilerParams(
            dimension_semantics=("parallel", "arbitrary")),
        name="dequant_codebook",
    )(codebook, idx)

    b, s, _ = x.shape
    m = b * s
    xm = x.reshape(m, _IN)
    m_tiles = m // _BM // 2

    out = pl.pallas_call(
        _matmul_body,
        grid=(2, m_tiles),
        in_specs=[
            pl.BlockSpec((_BM, _IN), lambda c, mm: (c * m_tiles + mm, 0)),
            pl.BlockSpec(memory_space=pl.ANY),
            pl.BlockSpec((1, _OUT), lambda c, mm: (0, 0)),
        ],
        out_specs=pl.BlockSpec((_BM, _OUT), lambda c, mm: (c * m_tiles + mm, 0)),
        out_shape=jax.ShapeDtypeStruct((m, _OUT), jnp.float32),
        scratch_shapes=[
            pltpu.VMEM((_OUT, _IN), jnp.bfloat16),
            pltpu.SemaphoreType.DMA,
        ],
        compiler_params=pltpu.CompilerParams(
            dimension_semantics=("parallel", "arbitrary")),
        name="dequant_linear_matmul",
    )(xm, w, bias.reshape(1, _OUT))
    return out.reshape(b, s, _OUT)


# speedup vs baseline: 1.1183x; 1.0399x over previous
import jax
import jax.numpy as jnp
from jax.experimental import pallas as pl
from jax.experimental.pallas import tpu as pltpu

_IN = 4096
_OUT = 4096
_CB = 256

_BN_DQ = 512
_BM = 256


def _dequant_body(cb_ref, idx_ref, w_ref):
    cb = cb_ref[...]
    idx = idx_ref[...]
    lo = cb[:, :128]
    hi = cb[:, 128:]
    glo = jnp.take_along_axis(lo, idx, axis=1)
    ghi = jnp.take_along_axis(hi, idx, axis=1)
    w = jnp.where(idx < 128, glo, ghi)
    w_ref[...] = w.astype(jnp.bfloat16)


def _matmul_body(x_ref, w_hbm, b_ref, o_ref, w_vmem, sem):
    @pl.when(pl.program_id(0) == 0)
    def _():
        pltpu.make_async_copy(w_hbm, w_vmem, sem).start()
        pltpu.make_async_copy(w_hbm, w_vmem, sem).wait()

    h = _IN // 2
    xa = x_ref[:, :h].astype(jnp.bfloat16)
    xb = x_ref[:, h:].astype(jnp.bfloat16)
    dn = (((1,), (1,)), ((), ()))
    acc = jax.lax.dot_general(
        xa, w_vmem[:, :h], dn, preferred_element_type=jnp.float32)
    acc = acc + jax.lax.dot_general(
        xb, w_vmem[:, h:], dn, preferred_element_type=jnp.float32)
    o_ref[...] = acc + b_ref[...]


def kernel(x, codebook, bias, indices):
    idx = indices.astype(jnp.int32)

    n_dq = _OUT // _BN_DQ
    n_dq_half = n_dq // 2
    w = pl.pallas_call(
        _dequant_body,
        grid=(2, n_dq_half),
        in_specs=[
            pl.BlockSpec((_BN_DQ, _CB), lambda c, n: (c * n_dq_half + n, 0)),
            pl.BlockSpec((_BN_DQ, _IN), lambda c, n: (c * n_dq_half + n, 0)),
        ],
        out_specs=pl.BlockSpec((_BN_DQ, _IN), lambda c, n: (c * n_dq_half + n, 0)),
        out_shape=jax.ShapeDtypeStruct((_OUT, _IN), jnp.bfloat16),
        compiler_params=pltpu.CompilerParams(
            dimension_semantics=("parallel", "arbitrary")),
        name="dequant_codebook",
    )(codebook, idx)

    b, s, _ = x.shape
    m = b * s
    xm = x.reshape(m, _IN)
    m_tiles = m // _BM

    out = pl.pallas_call(
        _matmul_body,
        grid=(m_tiles,),
        in_specs=[
            pl.BlockSpec((_BM, _IN), lambda mm: (mm, 0)),
            pl.BlockSpec(memory_space=pl.ANY),
            pl.BlockSpec((1, _OUT), lambda mm: (0, 0)),
        ],
        out_specs=pl.BlockSpec((_BM, _OUT), lambda mm: (mm, 0)),
        out_shape=jax.ShapeDtypeStruct((m, _OUT), jnp.float32),
        scratch_shapes=[
            pltpu.VMEM((_OUT, _IN), jnp.bfloat16),
            pltpu.SemaphoreType.DMA,
        ],
        compiler_params=pltpu.CompilerParams(
            dimension_semantics=("arbitrary",)),
        name="dequant_linear_matmul",
    )(xm, w, bias.reshape(1, _OUT))
    return out.reshape(b, s, _OUT)
